# baseline (speedup 1.0000x reference)
import functools
import math

import jax
import jax.numpy as jnp
from jax import lax
from jax.experimental import pallas as pl
from jax.experimental.pallas import tpu as pltpu
from jax.experimental.pallas import tpu_sc as plsc

N = 10000
E = 320000
D = 128
NP = 10240
NTILES = 32
EPT = E // NTILES
GROUPS = EPT // 16
RPT = NP // 16
BG = 80
NB = EPT // BG
BN = 2000


_GDN = lax.GatherDimensionNumbers(offset_dims=(), collapsed_slice_dims=(0,),
                                  start_index_map=(0,))


def _sc_mesh():
    return plsc.VectorSubcoreMesh(core_axis_name="c", subcore_axis_name="s",
                                  num_cores=2, num_subcores=16)


_CA = 9984
_CL = E - 31 * _CA


@functools.cache
def _make_s1():
    return pl.kernel(
        _s1_body,
        out_type=[
            jax.ShapeDtypeStruct((2, 1, NP), jnp.float32),
            jax.ShapeDtypeStruct((E,), jnp.int32),
            jax.ShapeDtypeStruct((E,), jnp.int32),
        ],
        mesh=_sc_mesh(),
        scratch_types=[
            pltpu.VMEM_SHARED((16, 1, NP), jnp.float32),
            pltpu.VMEM((NP,), jnp.float32),
            pltpu.VMEM((_CL,), jnp.int32),
            pltpu.VMEM((_CL,), jnp.int32),
            pltpu.VMEM((_CL,), jnp.float32),
            pltpu.VMEM((16, NP // 16), jnp.float32),
            pltpu.SemaphoreType.DMA,
        ],
        compiler_params=pltpu.CompilerParams(needs_layout_passes=False),
    )


def _s1_body(ei_hbm, ew_hbm, out_hbm, rout_hbm, cout_hbm,
             stage_sh, deg_v, row_v, col_v, ew_v, red_v, sem):
    c = lax.axis_index("c")
    s = lax.axis_index("s")
    wid = c * 16 + s
    base = wid * _CA

    def stage(n):
        pltpu.async_copy(ei_hbm.at[0, pl.ds(base, n)], row_v.at[pl.ds(0, n)],
                         sem)
        pltpu.async_copy(ei_hbm.at[1, pl.ds(base, n)], col_v.at[pl.ds(0, n)],
                         sem)
        pltpu.async_copy(ew_hbm.at[pl.ds(base, n)], ew_v.at[pl.ds(0, n)], sem)

    @pl.when(wid < NTILES - 1)
    def _():
        stage(_CA)

    @pl.when(wid == NTILES - 1)
    def _():
        stage(_CL)

    def zero_body(i, _):
        for u in range(8):
            deg_v[pl.ds(i * 128 + u * 16, 16)] = jnp.zeros((16,), jnp.float32)
        return 0

    lax.fori_loop(0, NP // 128, zero_body, 0)

    def dr(n):
        for dst in (row_v, col_v, ew_v):
            pltpu.make_async_copy(ew_hbm.at[pl.ds(0, n)],
                                  dst.at[pl.ds(0, n)], sem).wait()

    def hist(lo, hi):
        def body(g, _):
            for u in range(4):
                sl = pl.ds(g * 64 + u * 16, 16)
                plsc.addupdate_scatter(deg_v, [col_v[sl]], ew_v[sl])
            return 0

        lax.fori_loop(lo, hi, body, 0)

    @pl.when(wid < NTILES - 1)
    def _():
        dr(_CA)
        pltpu.async_copy(row_v.at[pl.ds(0, _CA)],
                         rout_hbm.at[pl.ds(base, _CA)], sem)
        pltpu.async_copy(col_v.at[pl.ds(0, _CA)],
                         cout_hbm.at[pl.ds(base, _CA)], sem)
        hist(0, _CA // 64)

    @pl.when(wid == NTILES - 1)
    def _():
        dr(_CL)
        pltpu.async_copy(row_v.at[pl.ds(0, _CL)],
                         rout_hbm.at[pl.ds(base, _CL)], sem)
        pltpu.async_copy(col_v.at[pl.ds(0, _CL)],
                         cout_hbm.at[pl.ds(base, _CL)], sem)
        hist(0, _CL // 64)

    pltpu.sync_copy(deg_v, stage_sh.at[s, 0])
    plsc.subcore_barrier()
    seg = NP // 16
    pltpu.sync_copy(stage_sh.at[:, 0, pl.ds(s * seg, seg)], red_v)

    def red_body(i, _):
        sl = pl.ds(i * 16, 16)
        acc = red_v[0, sl]
        for t in range(1, 16):
            acc = acc + red_v[t, sl]
        ew_v[sl] = acc
        return 0

    lax.fori_loop(0, seg // 16, red_body, 0)
    pltpu.sync_copy(ew_v.at[pl.ds(0, seg)],
                    out_hbm.at[c, 0, pl.ds(s * seg, seg)])

    @pl.when(wid < NTILES - 1)
    def _():
        for n in (_CA, _CA):
            pltpu.make_async_copy(ew_hbm.at[pl.ds(0, n)],
                                  row_v.at[pl.ds(0, n)], sem).wait()

    @pl.when(wid == NTILES - 1)
    def _():
        for n in (_CL, _CL):
            pltpu.make_async_copy(ew_hbm.at[pl.ds(0, n)],
                                  row_v.at[pl.ds(0, n)], sem).wait()


@functools.cache
def _make_s3():
    return pl.kernel(
        _s3_body,
        out_type=jax.ShapeDtypeStruct((2, NP, D), jnp.float32),
        mesh=_sc_mesh(),
        scratch_types=(
            [pltpu.VMEM_SHARED((NP, D), jnp.float32)]
            + [pltpu.VMEM((BG,), jnp.int32) for _ in range(4)]
            + [pltpu.VMEM((BG,), jnp.int32) for _ in range(4)]
            + [pltpu.VMEM((BG,), jnp.float32) for _ in range(4)]
            + [
                pltpu.VMEM((BG, D), jnp.float32),
                pltpu.VMEM((BG, D), jnp.float32),
                pltpu.VMEM((BG, D), jnp.float32),
                pltpu.VMEM((BG, D), jnp.float32),
                pltpu.SemaphoreType.DMA,
                pltpu.SemaphoreType.DMA,
                pltpu.SemaphoreType.DMA,
            ]
        ),
        compiler_params=pltpu.CompilerParams(needs_layout_passes=False),
    )


def _s3_body(row_hbm, col_hbm, ew_hbm, g_hbm, out_hbm,
             acc_sh, er0, er1, er2, er3, ec0, ec1, ec2, ec3,
             ef0, ef1, ef2, ef3, rows_a, rows_b, msg_a, msg_b,
             esem, gsem, ssem):
    erow = [er0, er1, er2, er3]
    ecol = [ec0, ec1, ec2, ec3]
    eew = [ef0, ef1, ef2, ef3]
    rows = [rows_a, rows_b]
    msg = [msg_a, msg_b]
    c = lax.axis_index("c")
    s = lax.axis_index("s")
    wid = c * 16 + s
    bbase = wid * NB

    def load_batch(b, slot, sem):
        off = b * BG
        pltpu.async_copy(row_hbm.at[pl.ds(off, BG)], erow[slot], sem)
        pltpu.async_copy(col_hbm.at[pl.ds(off, BG)], ecol[slot], sem)
        pltpu.async_copy(ew_hbm.at[pl.ds(off, BG)], eew[slot], sem)

    def drain(sem, dst):
        pltpu.make_async_copy(g_hbm.at[pl.ds(0, dst.shape[0])]
                              if dst.ndim == 2 else row_hbm.at[pl.ds(0, BG)],
                              dst, sem).wait()

    def compute(slot, rp, mp):
        rbuf = rows[rp]
        mbuf = msg[mp]
        for sub in range(BG // 16):
            sl16 = pl.ds(sub * 16, 16)
            svals = eew[slot][sl16]
            for e in range(16):
                se = svals[e]
                er = sub * 16 + e
                for j in range(D // 16):
                    sl = pl.ds(j * 16, 16)
                    mbuf[er, sl] = rbuf[er, sl] * se

    load_batch(bbase, 0, esem)
    load_batch(bbase + 1, 1, esem)

    def mzero(i, _):
        for j in range(D // 16):
            msg_a[i, pl.ds(j * 16, 16)] = jnp.zeros((16,), jnp.float32)
        return 0

    lax.fori_loop(0, BG, mzero, 0)
    for k in range(RPT // BG):
        pltpu.async_copy(msg_a, acc_sh.at[pl.ds(s * RPT + k * BG, BG)], ssem)
    for _ in range(3):
        drain(esem, er0)
    pltpu.async_copy(g_hbm.at[erow[0]], rows[0], gsem)
    for _ in range(3):
        drain(esem, er1)
    pltpu.async_copy(g_hbm.at[erow[1]], rows[1], gsem)
    load_batch(bbase + 2, 2, esem)
    for _ in range(RPT // BG):
        drain(ssem, msg_a)
    plsc.subcore_barrier()
    drain(gsem, rows[0])
    compute(0, 0, 0)
    pltpu.async_copy(msg[0], acc_sh.at[ecol[0]], ssem, add=True)

    def body(k, _):
        for u in range(4):
            b = 4 * k + u + 1
            slot = (u + 1) % 4
            nslot = (u + 2) % 4
            lslot = (u + 3) % 4
            rp = (u + 1) % 2
            mp = (u + 1) % 2

            @pl.when(b <= NB - 2)
            def _():
                for _ in range(3):
                    drain(esem, erow[nslot])
                pltpu.async_copy(g_hbm.at[erow[nslot]], rows[(rp + 1) % 2],
                                 gsem)

            @pl.when(b <= NB - 3)
            def _():
                load_batch(bbase + b + 2, lslot, esem)

            drain(gsem, rows[rp])

            @pl.when(b >= 2)
            def _():
                drain(ssem, msg[mp])
            compute(slot, rp, mp)
            pltpu.async_copy(msg[mp], acc_sh.at[ecol[slot]], ssem, add=True)
        return 0

    lax.fori_loop(0, (NB - 1) // 4, body, 0)
    drain(ssem, msg_a)
    drain(ssem, msg_b)
    plsc.subcore_barrier()
    pltpu.sync_copy(acc_sh.at[pl.ds(s * RPT, RPT)],
                    out_hbm.at[c, pl.ds(s * RPT, RPT)])


def _tc0_body(x_ref, g_ref, be_ref, w_ref, dpt_ref, gout_ref, dis_ref):
    x = x_ref[...]
    mu = jnp.mean(x, axis=1, keepdims=True)
    xc = x - mu
    var = jnp.mean(xc * xc, axis=1, keepdims=True)
    xn = xc * lax.rsqrt(var + 1e-5) * g_ref[...] + be_ref[...]
    h = jnp.dot(xn.astype(jnp.bfloat16), w_ref[...].astype(jnp.bfloat16),
                preferred_element_type=jnp.float32)
    deg = 1.0 + jnp.sum(dpt_ref[...], axis=1, keepdims=True)
    pos = deg > 0
    dis = jnp.where(pos, lax.rsqrt(jnp.where(pos, deg, 1.0)), 0.0)
    gout_ref[...] = h * dis
    dis_ref[...] = dis


def _tc0(x, g2, be2, W, deg_part_t):
    return pl.pallas_call(
        _tc0_body,
        grid=(N // BN,),
        in_specs=[
            pl.BlockSpec((BN, D), lambda i: (i, 0)),
            pl.BlockSpec((1, D), lambda i: (0, 0)),
            pl.BlockSpec((1, D), lambda i: (0, 0)),
            pl.BlockSpec((D, D), lambda i: (0, 0)),
            pl.BlockSpec((BN, 2), lambda i: (i, 0)),
        ],
        out_specs=[
            pl.BlockSpec((BN, D), lambda i: (i, 0)),
            pl.BlockSpec((BN, 1), lambda i: (i, 0)),
        ],
        out_shape=[
            jax.ShapeDtypeStruct((N, D), jnp.float32),
            jax.ShapeDtypeStruct((N, 1), jnp.float32),
        ],
    )(x, g2, be2, W, deg_part_t)


_INV_SQRT2 = 1.0 / math.sqrt(2.0)


def _tc2_body(acc_ref, g_ref, dis_ref, b_ref, out_ref):
    a = acc_ref[0] + acc_ref[1] + g_ref[...]
    pre = dis_ref[...] * a + b_ref[...]
    out_ref[...] = 0.5 * pre * (1.0 + lax.erf(pre * _INV_SQRT2))


def _tc2(acc, g, dis, b2):
    return pl.pallas_call(
        _tc2_body,
        grid=(N // BN,),
        in_specs=[
            pl.BlockSpec((2, BN, D), lambda i: (0, i, 0)),
            pl.BlockSpec((BN, D), lambda i: (i, 0)),
            pl.BlockSpec((BN, 1), lambda i: (i, 0)),
            pl.BlockSpec((1, D), lambda i: (0, 0)),
        ],
        out_specs=pl.BlockSpec((BN, D), lambda i: (i, 0)),
        out_shape=jax.ShapeDtypeStruct((N, D), jnp.float32),
    )(acc, g, dis, b2)


def kernel(x, edge_index, edge_weight, ln_gamma, ln_beta, W, b):
    deg_part, row, col = _make_s1()(edge_index, edge_weight)
    g, dis = _tc0(x, ln_gamma.reshape(1, D), ln_beta.reshape(1, D), W,
                  deg_part.reshape(2, NP).T)
    acc = _make_s3()(row, col, edge_weight, g)
    return _tc2(acc, g, dis, b.reshape(1, D))


# speedup vs baseline: 1.0050x; 1.0050x over previous
import functools
import math

import jax
import jax.numpy as jnp
from jax import lax
from jax.experimental import pallas as pl
from jax.experimental.pallas import tpu as pltpu
from jax.experimental.pallas import tpu_sc as plsc

N = 10000
E = 320000
D = 128
NP = 10240
NTILES = 32
EPT = E // NTILES
RPT = NP // 16
BG = 80
NB = EPT // BG
BN = 2000


def _sc_mesh():
    return plsc.VectorSubcoreMesh(core_axis_name="c", subcore_axis_name="s",
                                  num_cores=2, num_subcores=16)


_CA = 9984
_CL = E - 31 * _CA


@functools.cache
def _make_s1():
    return pl.kernel(
        _s1_body,
        out_type=[
            jax.ShapeDtypeStruct((2, 1, NP), jnp.float32),
            jax.ShapeDtypeStruct((E,), jnp.int32),
            jax.ShapeDtypeStruct((E,), jnp.int32),
        ],
        mesh=_sc_mesh(),
        scratch_types=[
            pltpu.VMEM_SHARED((16, 1, NP), jnp.float32),
            pltpu.VMEM((NP,), jnp.float32),
            pltpu.VMEM((_CL,), jnp.int32),
            pltpu.VMEM((_CL,), jnp.int32),
            pltpu.VMEM((_CL,), jnp.float32),
            pltpu.VMEM((16, NP // 16), jnp.float32),
            pltpu.SemaphoreType.DMA,
        ],
        compiler_params=pltpu.CompilerParams(needs_layout_passes=False),
    )


def _s1_body(ei_hbm, ew_hbm, out_hbm, rout_hbm, cout_hbm,
             stage_sh, deg_v, row_v, col_v, ew_v, red_v, sem):
    c = lax.axis_index("c")
    s = lax.axis_index("s")
    wid = c * 16 + s
    base = wid * _CA

    def stage(n):
        pltpu.async_copy(ei_hbm.at[0, pl.ds(base, n)], row_v.at[pl.ds(0, n)],
                         sem)
        pltpu.async_copy(ei_hbm.at[1, pl.ds(base, n)], col_v.at[pl.ds(0, n)],
                         sem)
        pltpu.async_copy(ew_hbm.at[pl.ds(base, n)], ew_v.at[pl.ds(0, n)], sem)

    @pl.when(wid < NTILES - 1)
    def _():
        stage(_CA)

    @pl.when(wid == NTILES - 1)
    def _():
        stage(_CL)

    def zero_body(i, _):
        for u in range(8):
            deg_v[pl.ds(i * 128 + u * 16, 16)] = jnp.zeros((16,), jnp.float32)
        return 0

    lax.fori_loop(0, NP // 128, zero_body, 0)

    def dr(n):
        for dst in (row_v, col_v, ew_v):
            pltpu.make_async_copy(ew_hbm.at[pl.ds(0, n)],
                                  dst.at[pl.ds(0, n)], sem).wait()

    def hist(lo, hi):
        def body(g, _):
            for u in range(4):
                sl = pl.ds(g * 64 + u * 16, 16)
                plsc.addupdate_scatter(deg_v, [col_v[sl]], ew_v[sl])
            return 0

        lax.fori_loop(lo, hi, body, 0)

    @pl.when(wid < NTILES - 1)
    def _():
        dr(_CA)
        pltpu.async_copy(row_v.at[pl.ds(0, _CA)],
                         rout_hbm.at[pl.ds(base, _CA)], sem)
        pltpu.async_copy(col_v.at[pl.ds(0, _CA)],
                         cout_hbm.at[pl.ds(base, _CA)], sem)
        hist(0, _CA // 64)

    @pl.when(wid == NTILES - 1)
    def _():
        dr(_CL)
        pltpu.async_copy(row_v.at[pl.ds(0, _CL)],
                         rout_hbm.at[pl.ds(base, _CL)], sem)
        pltpu.async_copy(col_v.at[pl.ds(0, _CL)],
                         cout_hbm.at[pl.ds(base, _CL)], sem)
        hist(0, _CL // 64)

    pltpu.sync_copy(deg_v, stage_sh.at[s, 0])
    plsc.subcore_barrier()
    seg = NP // 16
    pltpu.sync_copy(stage_sh.at[:, 0, pl.ds(s * seg, seg)], red_v)

    def red_body(i, _):
        sl = pl.ds(i * 16, 16)
        acc = red_v[0, sl]
        for t in range(1, 16):
            acc = acc + red_v[t, sl]
        ew_v[sl] = acc
        return 0

    lax.fori_loop(0, seg // 16, red_body, 0)
    pltpu.sync_copy(ew_v.at[pl.ds(0, seg)],
                    out_hbm.at[c, 0, pl.ds(s * seg, seg)])

    @pl.when(wid < NTILES - 1)
    def _():
        for n in (_CA, _CA):
            pltpu.make_async_copy(ew_hbm.at[pl.ds(0, n)],
                                  row_v.at[pl.ds(0, n)], sem).wait()

    @pl.when(wid == NTILES - 1)
    def _():
        for n in (_CL, _CL):
            pltpu.make_async_copy(ew_hbm.at[pl.ds(0, n)],
                                  row_v.at[pl.ds(0, n)], sem).wait()


@functools.cache
def _make_s3():
    return pl.kernel(
        _s3_body,
        out_type=jax.ShapeDtypeStruct((2, NP, D), jnp.float32),
        mesh=_sc_mesh(),
        scratch_types=(
            [pltpu.VMEM_SHARED((NP, D), jnp.float32)]
            + [pltpu.VMEM((BG,), jnp.int32) for _ in range(4)]
            + [pltpu.VMEM((BG,), jnp.int32) for _ in range(4)]
            + [pltpu.VMEM((BG,), jnp.float32) for _ in range(4)]
            + [
                pltpu.VMEM((BG, D), jnp.float32),
                pltpu.VMEM((BG, D), jnp.float32),
                pltpu.VMEM((BG, D), jnp.float32),
                pltpu.VMEM((BG, D), jnp.float32),
                pltpu.SemaphoreType.DMA,
                pltpu.SemaphoreType.DMA,
                pltpu.SemaphoreType.DMA,
            ]
        ),
        compiler_params=pltpu.CompilerParams(needs_layout_passes=False),
    )


def _s3_body(row_hbm, col_hbm, ew_hbm, g_hbm, out_hbm,
             acc_sh, er0, er1, er2, er3, ec0, ec1, ec2, ec3,
             ef0, ef1, ef2, ef3, rows_a, rows_b, msg_a, msg_b,
             esem, gsem, ssem):
    erow = [er0, er1, er2, er3]
    ecol = [ec0, ec1, ec2, ec3]
    eew = [ef0, ef1, ef2, ef3]
    rows = [rows_a, rows_b]
    msg = [msg_a, msg_b]
    c = lax.axis_index("c")
    s = lax.axis_index("s")
    wid = c * 16 + s
    bbase = wid * NB

    def load_batch(b, slot, sem):
        off = b * BG
        pltpu.async_copy(row_hbm.at[pl.ds(off, BG)], erow[slot], sem)
        pltpu.async_copy(col_hbm.at[pl.ds(off, BG)], ecol[slot], sem)
        pltpu.async_copy(ew_hbm.at[pl.ds(off, BG)], eew[slot], sem)

    def drain(sem, dst):
        pltpu.make_async_copy(g_hbm.at[pl.ds(0, dst.shape[0])]
                              if dst.ndim == 2 else row_hbm.at[pl.ds(0, BG)],
                              dst, sem).wait()

    def compute(slot, rp, mp):
        rbuf = rows[rp]
        mbuf = msg[mp]
        for sub in range(BG // 16):
            sl16 = pl.ds(sub * 16, 16)
            svals = eew[slot][sl16]
            for e in range(16):
                se = svals[e]
                er = sub * 16 + e
                for j in range(D // 16):
                    sl = pl.ds(j * 16, 16)
                    mbuf[er, sl] = rbuf[er, sl] * se

    load_batch(bbase, 0, esem)
    load_batch(bbase + 1, 1, esem)

    def mzero(i, _):
        for j in range(D // 16):
            msg_a[i, pl.ds(j * 16, 16)] = jnp.zeros((16,), jnp.float32)
        return 0

    lax.fori_loop(0, BG, mzero, 0)
    for k in range(RPT // BG):
        pltpu.async_copy(msg_a, acc_sh.at[pl.ds(s * RPT + k * BG, BG)], ssem)
    for _ in range(3):
        drain(esem, er0)
    pltpu.async_copy(g_hbm.at[erow[0]], rows[0], gsem)
    for _ in range(3):
        drain(esem, er1)
    pltpu.async_copy(g_hbm.at[erow[1]], rows[1], gsem)
    load_batch(bbase + 2, 2, esem)
    for _ in range(RPT // BG):
        drain(ssem, msg_a)
    plsc.subcore_barrier()
    drain(gsem, rows[0])
    compute(0, 0, 0)
    pltpu.async_copy(msg[0], acc_sh.at[ecol[0]], ssem, add=True)

    def body(k, _):
        for u in range(4):
            b = 4 * k + u + 1
            slot = (u + 1) % 4
            nslot = (u + 2) % 4
            lslot = (u + 3) % 4
            rp = (u + 1) % 2
            mp = (u + 1) % 2

            @pl.when(b <= NB - 2)
            def _():
                for _ in range(3):
                    drain(esem, erow[nslot])
                pltpu.async_copy(g_hbm.at[erow[nslot]], rows[(rp + 1) % 2],
                                 gsem)

            @pl.when(b <= NB - 3)
            def _():
                load_batch(bbase + b + 2, lslot, esem)

            drain(gsem, rows[rp])

            @pl.when(b >= 2)
            def _():
                drain(ssem, msg[mp])
            compute(slot, rp, mp)
            pltpu.async_copy(msg[mp], acc_sh.at[ecol[slot]], ssem, add=True)
        return 0

    lax.fori_loop(0, (NB - 1) // 4, body, 0)
    drain(ssem, msg_a)
    drain(ssem, msg_b)
    plsc.subcore_barrier()
    pltpu.sync_copy(acc_sh.at[pl.ds(s * RPT, RPT)],
                    out_hbm.at[c, pl.ds(s * RPT, RPT)])


def _tc0_body(x_ref, g_ref, be_ref, w_ref, dpt_ref, gout_ref, dis_ref):
    x = x_ref[...]
    mu = jnp.mean(x, axis=1, keepdims=True)
    xc = x - mu
    var = jnp.mean(xc * xc, axis=1, keepdims=True)
    xn = xc * lax.rsqrt(var + 1e-5) * g_ref[...] + be_ref[...]
    h = jnp.dot(xn.astype(jnp.bfloat16), w_ref[...].astype(jnp.bfloat16),
                preferred_element_type=jnp.float32)
    deg = 1.0 + jnp.sum(dpt_ref[...], axis=1, keepdims=True)
    pos = deg > 0
    dis = jnp.where(pos, lax.rsqrt(jnp.where(pos, deg, 1.0)), 0.0)
    gout_ref[...] = h * dis
    dis_ref[...] = dis


def _tc0(x, g2, be2, W, deg_part_t):
    return pl.pallas_call(
        _tc0_body,
        grid=(N // BN,),
        in_specs=[
            pl.BlockSpec((BN, D), lambda i: (i, 0)),
            pl.BlockSpec((1, D), lambda i: (0, 0)),
            pl.BlockSpec((1, D), lambda i: (0, 0)),
            pl.BlockSpec((D, D), lambda i: (0, 0)),
            pl.BlockSpec((BN, 2), lambda i: (i, 0)),
        ],
        out_specs=[
            pl.BlockSpec((BN, D), lambda i: (i, 0)),
            pl.BlockSpec((BN, 1), lambda i: (i, 0)),
        ],
        out_shape=[
            jax.ShapeDtypeStruct((N, D), jnp.float32),
            jax.ShapeDtypeStruct((N, 1), jnp.float32),
        ],
    )(x, g2, be2, W, deg_part_t)


_INV_SQRT2 = 1.0 / math.sqrt(2.0)


def _tc2_body(acc_ref, g_ref, dis_ref, b_ref, out_ref):
    a = acc_ref[0] + acc_ref[1] + g_ref[...]
    pre = dis_ref[...] * a + b_ref[...]
    out_ref[...] = 0.5 * pre * (1.0 + lax.erf(pre * _INV_SQRT2))


def _tc2(acc, g, dis, b2):
    return pl.pallas_call(
        _tc2_body,
        grid=(N // BN,),
        in_specs=[
            pl.BlockSpec((2, BN, D), lambda i: (0, i, 0)),
            pl.BlockSpec((BN, D), lambda i: (i, 0)),
            pl.BlockSpec((BN, 1), lambda i: (i, 0)),
            pl.BlockSpec((1, D), lambda i: (0, 0)),
        ],
        out_specs=pl.BlockSpec((BN, D), lambda i: (i, 0)),
        out_shape=jax.ShapeDtypeStruct((N, D), jnp.float32),
    )(acc, g, dis, b2)


def kernel(x, edge_index, edge_weight, ln_gamma, ln_beta, W, b):
    deg_part, row, col = _make_s1()(edge_index, edge_weight)
    g, dis = _tc0(x, ln_gamma.reshape(1, D), ln_beta.reshape(1, D), W,
                  deg_part.reshape(2, NP).T)
    acc = _make_s3()(row, col, edge_weight, g)
    return _tc2(acc, g, dis, b.reshape(1, D))
